# baseline (speedup 1.0000x reference)
import functools

import jax
import jax.numpy as jnp
from jax import lax
from jax.experimental import pallas as pl
from jax.experimental.pallas import tpu as pltpu
from jax.experimental.pallas import tpu_sc as plsc

VOCAB = 1000000
EMB = 32
BATCH = 16384
HIST = 50

NC = 2
NS = 16
NW = NC * NS
L = 16

BAT_PER_W = BATCH // NW
TB = 128
NTB = BAT_PER_W // TB
NUNIT = HIST * NTB
NBUF = 3
WROW = 128
EPR = WROW // EMB


def _make_kernel():
  mesh = plsc.VectorSubcoreMesh(
      core_axis_name="c", subcore_axis_name="s", num_cores=NC, num_subcores=NS
  )

  @functools.partial(
      pl.kernel,
      out_type=jax.ShapeDtypeStruct((HIST, EMB, BATCH), jnp.float32),
      mesh=mesh,
      scratch_types=[
          pltpu.VMEM((HIST, BAT_PER_W), jnp.int32),
          [pltpu.VMEM((TB,), jnp.int32) for _ in range(NBUF)],
          [pltpu.VMEM((TB, WROW), jnp.float32) for _ in range(NBUF)],
          [pltpu.VMEM((EMB, TB + 1), jnp.float32) for _ in range(NBUF)],
          [pltpu.SemaphoreType.DMA for _ in range(NBUF)],
          [pltpu.SemaphoreType.DMA for _ in range(NBUF)],
      ],
      compiler_params=pltpu.CompilerParams(
          use_tc_tiling_on_sc=True,
          needs_layout_passes=False,
          disable_bounds_checks=True,
      ),
  )
  def gather_kernel(xt_hbm, w4_hbm, out_hbm, idxt_v, gidx, rows, tbufs,
                    gsems, wsems):
    wid = lax.axis_index("s") * NC + lax.axis_index("c")
    bat0 = wid * BAT_PER_W

    iota = lax.iota(jnp.int32, L)
    e0vec = iota
    e1vec = iota + L

    pltpu.sync_copy(xt_hbm.at[:, pl.ds(bat0, BAT_PER_W)], idxt_v)

    def start_g(u, b):
      h = u // NTB
      tb = lax.rem(u, NTB)
      for c in range(TB // L):
        v = idxt_v[h, pl.ds(tb * TB + c * L, L)]
        gidx[b][pl.ds(c * L, L)] = lax.shift_right_logical(v, 2)
      pltpu.async_copy(w4_hbm.at[gidx[b]], rows[b], gsems[b])

    def wait_g(b):
      pltpu.make_async_copy(w4_hbm.at[pl.ds(0, TB)], rows[b], gsems[b]).wait()

    def transpose(u, b):
      h = u // NTB
      tb = lax.rem(u, NTB)
      zero = jnp.zeros((L,), jnp.int32)

      @pl.loop(0, TB // L)
      def _(c):
        idxv = idxt_v[h, pl.ds(tb * TB + c * L, L)]
        offv = lax.shift_left(lax.bitwise_and(idxv, 3), 5)
        for j in range(L):
          r = c * L + j
          rvec = r + zero
          off = offv[j]
          lo = rows[b][r, pl.ds(off, L)]
          hi = rows[b][r, pl.ds(off + L, L)]
          plsc.store_scatter(tbufs[b], [e0vec, rvec], lo)
          plsc.store_scatter(tbufs[b], [e1vec, rvec], hi)

    def _dst(u):
      h = u // NTB
      tb = lax.rem(u, NTB)
      return out_hbm.at[h, :, pl.ds(bat0 + tb * TB, TB)]

    def start_w(u, b):
      pltpu.async_copy(tbufs[b].at[:, pl.ds(0, TB)], _dst(u), wsems[b])

    def wait_w(u, b):
      pltpu.make_async_copy(
          tbufs[b].at[:, pl.ds(0, TB)], _dst(u), wsems[b]
      ).wait()

    start_g(0, 0)
    start_g(1, 1)
    start_g(2, 2)
    wait_g(0)
    transpose(0, 0)
    start_w(0, 0)

    @pl.loop(3, 198, step=NBUF)
    def _(u0):
      for j in range(NBUF):
        u = u0 + j
        b = j
        b2 = (j + 1) % NBUF
        wait_w(u - NBUF, b)
        start_g(u, b)
        wait_g(b2)
        transpose(u - 2, b2)
        start_w(u - 2, b2)

    wait_w(195, 0)
    start_g(198, 0)
    wait_g(1)
    transpose(196, 1)
    start_w(196, 1)
    wait_w(196, 1)
    start_g(199, 1)
    wait_g(2)
    transpose(197, 2)
    start_w(197, 2)
    wait_g(0)
    transpose(198, 0)
    start_w(198, 0)
    wait_g(1)
    transpose(199, 1)
    start_w(199, 1)
    wait_w(197, 2)
    wait_w(198, 0)
    wait_w(199, 1)

  return gather_kernel


_kernel_call = _make_kernel()


@jax.jit
def kernel(x, W):
  xt = jnp.transpose(x.astype(jnp.int32))
  w4 = W.reshape(VOCAB * EMB // WROW, WROW)
  out_t = _kernel_call(xt, w4)
  return jnp.transpose(out_t, (2, 0, 1))


# speedup vs baseline: 1.1540x; 1.1540x over previous
import functools

import jax
import jax.numpy as jnp
from jax import lax
from jax.experimental import pallas as pl
from jax.experimental.pallas import tpu as pltpu
from jax.experimental.pallas import tpu_sc as plsc

VOCAB = 1000000
EMB = 32
BATCH = 16384
HIST = 50

NC = 2
NS = 16
NW = NC * NS
L = 16

BAT_PER_W = BATCH // NW
TB = 128
NTB = BAT_PER_W // TB
NUNIT = HIST * NTB
NBUF = 4


def _make_kernel():
  mesh = plsc.VectorSubcoreMesh(
      core_axis_name="c", subcore_axis_name="s", num_cores=NC, num_subcores=NS
  )

  @functools.partial(
      pl.kernel,
      out_type=jax.ShapeDtypeStruct((HIST, EMB, BATCH), jnp.float32),
      mesh=mesh,
      scratch_types=[
          pltpu.VMEM((BAT_PER_W, HIST), jnp.int32),
          pltpu.VMEM((HIST, BAT_PER_W), jnp.int32),
          [pltpu.VMEM((TB, EMB), jnp.float32) for _ in range(NBUF)],
          [pltpu.VMEM((EMB, TB + 1), jnp.float32) for _ in range(NBUF)],
          [pltpu.SemaphoreType.DMA for _ in range(NBUF)],
          [pltpu.SemaphoreType.DMA for _ in range(NBUF)],
      ],
      compiler_params=pltpu.CompilerParams(
          use_tc_tiling_on_sc=False,
          needs_layout_passes=False,
          disable_bounds_checks=True,
      ),
  )
  def gather_kernel(x_hbm, w_hbm, out_hbm, idx_v, idxt_v, rows, tbufs,
                    gsems, wsems):
    wid = lax.axis_index("s") * NC + lax.axis_index("c")
    bat0 = wid * BAT_PER_W

    iota = lax.iota(jnp.int32, L)
    e0vec = iota
    e1vec = iota + L

    pltpu.sync_copy(x_hbm.at[pl.ds(bat0, BAT_PER_W)], idx_v)

    @pl.loop(0, HIST)
    def _(h):
      hvec = h + jnp.zeros((L,), jnp.int32)
      for c in range(BAT_PER_W // L):
        col = plsc.load_gather(idx_v, [iota + (c * L), hvec])
        idxt_v[h, pl.ds(c * L, L)] = col

    def start_g(u, b):
      h = u // NTB
      tb = lax.rem(u, NTB)
      pltpu.async_copy(
          w_hbm.at[idxt_v.at[h, pl.ds(tb * TB, TB)]], rows[b], gsems[b]
      )

    def wait_g(b):
      pltpu.make_async_copy(w_hbm.at[pl.ds(0, TB)], rows[b], gsems[b]).wait()

    def transpose(b):
      for r0 in range(0, TB, 8):
        vals = []
        for r in range(r0, r0 + 8):
          lo = rows[b][r, pl.ds(0, L)]
          hi = rows[b][r, pl.ds(L, L)]
          vals.append((r, lo, hi))
        for r, lo, hi in vals:
          rvec = jnp.full((L,), r, jnp.int32)
          plsc.store_scatter(tbufs[b], [e0vec, rvec], lo)
          plsc.store_scatter(tbufs[b], [e1vec, rvec], hi)

    def _dst(u):
      h = u // NTB
      tb = lax.rem(u, NTB)
      return out_hbm.at[h, :, pl.ds(bat0 + tb * TB, TB)]

    def start_w(u, b):
      pltpu.async_copy(tbufs[b].at[:, pl.ds(0, TB)], _dst(u), wsems[b])

    def wait_w(u, b):
      pltpu.make_async_copy(
          tbufs[b].at[:, pl.ds(0, TB)], _dst(u), wsems[b]
      ).wait()

    start_g(0, 0)
    start_g(1, 1)
    start_g(2, 2)
    wait_g(0)
    transpose(0)
    start_w(0, 0)
    start_g(3, 3)
    wait_g(1)
    transpose(1)
    start_w(1, 1)

    @pl.loop(4, NUNIT - NUNIT % NBUF, step=NBUF)
    def _(u0):
      for j in range(NBUF):
        u = u0 + j
        b = j
        b2 = (j + 2) % NBUF
        wait_w(u - NBUF, b)
        start_g(u, b)
        wait_g(b2)
        transpose(b2)
        start_w(u - 2, b2)

    wait_g(2)
    transpose(2)
    start_w(198, 2)
    wait_g(3)
    transpose(3)
    start_w(199, 3)
    wait_w(196, 0)
    wait_w(197, 1)
    wait_w(198, 2)
    wait_w(199, 3)

  return gather_kernel


_kernel_call = _make_kernel()


@jax.jit
def kernel(x, W):
  out_t = _kernel_call(x.astype(jnp.int32), W)
  return jnp.transpose(out_t, (2, 0, 1))


# speedup vs baseline: 1.1710x; 1.0148x over previous
import functools

import jax
import jax.numpy as jnp
from jax import lax
from jax.experimental import pallas as pl
from jax.experimental.pallas import tpu as pltpu
from jax.experimental.pallas import tpu_sc as plsc

VOCAB = 1000000
EMB = 32
BATCH = 16384
HIST = 50

NC = 2
NS = 16
NW = NC * NS
L = 16

BAT_PER_W = BATCH // NW
TB = 128
NTB = BAT_PER_W // TB
NUNIT = HIST * NTB
NBUF = 4


def _make_kernel():
  mesh = plsc.VectorSubcoreMesh(
      core_axis_name="c", subcore_axis_name="s", num_cores=NC, num_subcores=NS
  )

  @functools.partial(
      pl.kernel,
      out_type=jax.ShapeDtypeStruct((HIST, EMB, BATCH), jnp.float32),
      mesh=mesh,
      scratch_types=[
          pltpu.VMEM((BAT_PER_W, HIST), jnp.int32),
          pltpu.VMEM((HIST, BAT_PER_W), jnp.int32),
          [pltpu.VMEM((TB, EMB), jnp.float32) for _ in range(NBUF)],
          [pltpu.VMEM((EMB, TB + 1), jnp.float32) for _ in range(NBUF)],
          [pltpu.SemaphoreType.DMA for _ in range(NBUF)],
          [pltpu.SemaphoreType.DMA for _ in range(NBUF)],
      ],
      compiler_params=pltpu.CompilerParams(
          use_tc_tiling_on_sc=False,
          needs_layout_passes=False,
          disable_bounds_checks=True,
      ),
  )
  def gather_kernel(x_hbm, w_hbm, out_hbm, idx_v, idxt_v, rows, tbufs,
                    gsems, wsems):
    wid = lax.axis_index("s") * NC + lax.axis_index("c")
    bat0 = wid * BAT_PER_W

    iota = lax.iota(jnp.int32, L)
    e0vec = iota
    e1vec = iota + L

    pltpu.sync_copy(x_hbm.at[pl.ds(bat0, BAT_PER_W)], idx_v)

    @pl.loop(0, HIST)
    def _(h):
      hvec = h + jnp.zeros((L,), jnp.int32)
      for c in range(BAT_PER_W // L):
        col = plsc.load_gather(idx_v, [iota + (c * L), hvec])
        idxt_v[h, pl.ds(c * L, L)] = col

    def start_g(u, b):
      h = u // NTB
      tb = lax.rem(u, NTB)
      pltpu.async_copy(
          w_hbm.at[idxt_v.at[h, pl.ds(tb * TB, TB)]], rows[b], gsems[b]
      )

    def wait_g(b):
      pltpu.make_async_copy(w_hbm.at[pl.ds(0, TB)], rows[b], gsems[b]).wait()

    def transpose(b):
      for r in range(TB):
        rvec = jnp.full((L,), r, jnp.int32)
        lo = rows[b][r, pl.ds(0, L)]
        hi = rows[b][r, pl.ds(L, L)]
        plsc.store_scatter(tbufs[b], [e0vec, rvec], lo)
        plsc.store_scatter(tbufs[b], [e1vec, rvec], hi)

    def _dst(u):
      h = u // NTB
      tb = lax.rem(u, NTB)
      return out_hbm.at[h, :, pl.ds(bat0 + tb * TB, TB)]

    def start_w(u, b):
      pltpu.async_copy(tbufs[b].at[:, pl.ds(0, TB)], _dst(u), wsems[b])

    def wait_w(u, b):
      pltpu.make_async_copy(
          tbufs[b].at[:, pl.ds(0, TB)], _dst(u), wsems[b]
      ).wait()

    start_g(0, 0)
    start_g(1, 1)
    start_g(2, 2)
    wait_g(0)
    transpose(0)
    start_w(0, 0)
    start_g(3, 3)
    wait_g(1)
    transpose(1)
    start_w(1, 1)

    @pl.loop(4, NUNIT - NUNIT % NBUF, step=NBUF)
    def _(u0):
      for j in range(NBUF):
        u = u0 + j
        b = j
        b2 = (j + 2) % NBUF
        wait_w(u - NBUF, b)
        start_g(u, b)
        wait_g(b2)
        transpose(b2)
        start_w(u - 2, b2)

    wait_g(2)
    transpose(2)
    start_w(198, 2)
    wait_g(3)
    transpose(3)
    start_w(199, 3)
    wait_w(196, 0)
    wait_w(197, 1)
    wait_w(198, 2)
    wait_w(199, 3)

  return gather_kernel


_kernel_call = _make_kernel()


@jax.jit
def kernel(x, W):
  out_t = _kernel_call(x.astype(jnp.int32), W)
  return jnp.transpose(out_t, (2, 0, 1))


# speedup vs baseline: 1.3921x; 1.1889x over previous
import functools

import jax
import jax.numpy as jnp
from jax import lax
from jax.experimental import pallas as pl
from jax.experimental.pallas import tpu as pltpu
from jax.experimental.pallas import tpu_sc as plsc

VOCAB = 1000000
EMB = 32
BATCH = 16384
HIST = 50

NC = 2
NS = 16
NW = NC * NS
L = 16

BAT_PER_W = BATCH // NW
TB = 128
NTB = BAT_PER_W // TB
NUNIT = HIST * NTB
NBUF = 4


def _make_kernel():
  mesh = plsc.VectorSubcoreMesh(
      core_axis_name="c", subcore_axis_name="s", num_cores=NC, num_subcores=NS
  )

  @functools.partial(
      pl.kernel,
      out_type=jax.ShapeDtypeStruct((HIST, EMB, BATCH), jnp.float32),
      mesh=mesh,
      scratch_types=[
          pltpu.VMEM((BAT_PER_W, HIST), jnp.int32),
          pltpu.VMEM((HIST, BAT_PER_W), jnp.int32),
          [pltpu.VMEM((TB, EMB), jnp.float32) for _ in range(NBUF)],
          [pltpu.VMEM((EMB, TB + 1), jnp.float32) for _ in range(NBUF)],
          [pltpu.SemaphoreType.DMA for _ in range(NBUF)],
          [pltpu.SemaphoreType.DMA for _ in range(NBUF)],
      ],
      compiler_params=pltpu.CompilerParams(
          use_tc_tiling_on_sc=False,
          needs_layout_passes=False,
          disable_bounds_checks=True,
      ),
  )
  def gather_kernel(x_hbm, w_hbm, out_hbm, idx_v, idxt_v, rows, tbufs,
                    gsems, wsems):
    wid = lax.axis_index("s") * NC + lax.axis_index("c")
    bat0 = wid * BAT_PER_W

    iota = lax.iota(jnp.int32, L)
    e0vec = iota
    e1vec = iota + L
    zero = jnp.zeros((L,), jnp.int32)

    pltpu.sync_copy(x_hbm.at[pl.ds(bat0, BAT_PER_W)], idx_v)

    @pl.loop(0, HIST)
    def _(h):
      hvec = h + jnp.zeros((L,), jnp.int32)
      for c in range(BAT_PER_W // L):
        col = plsc.load_gather(idx_v, [iota + (c * L), hvec])
        idxt_v[h, pl.ds(c * L, L)] = col

    def start_g(u, b):
      h = u // NTB
      tb = lax.rem(u, NTB)
      pltpu.async_copy(
          w_hbm.at[idxt_v.at[h, pl.ds(tb * TB, TB)]], rows[b], gsems[b]
      )

    def wait_g(b):
      pltpu.make_async_copy(w_hbm.at[pl.ds(0, TB)], rows[b], gsems[b]).wait()

    def transpose(b):
      @pl.loop(0, TB // 8)
      def _(g):
        for d in range(8):
          r = g * 8 + d
          rvec = r + zero
          lo = rows[b][r, pl.ds(0, L)]
          hi = rows[b][r, pl.ds(L, L)]
          plsc.store_scatter(tbufs[b], [e0vec, rvec], lo)
          plsc.store_scatter(tbufs[b], [e1vec, rvec], hi)

    def _dst(u):
      h = u // NTB
      tb = lax.rem(u, NTB)
      return out_hbm.at[h, :, pl.ds(bat0 + tb * TB, TB)]

    def start_w(u, b):
      pltpu.async_copy(tbufs[b].at[:, pl.ds(0, TB)], _dst(u), wsems[b])

    def wait_w(u, b):
      pltpu.make_async_copy(
          tbufs[b].at[:, pl.ds(0, TB)], _dst(u), wsems[b]
      ).wait()

    start_g(0, 0)
    start_g(1, 1)
    start_g(2, 2)
    wait_g(0)
    transpose(0)
    start_w(0, 0)
    start_g(3, 3)
    wait_g(1)
    transpose(1)
    start_w(1, 1)

    @pl.loop(4, NUNIT - NUNIT % NBUF, step=NBUF)
    def _(u0):
      for j in range(NBUF):
        u = u0 + j
        b = j
        b2 = (j + 2) % NBUF
        wait_w(u - NBUF, b)
        start_g(u, b)
        wait_g(b2)
        transpose(b2)
        start_w(u - 2, b2)

    wait_g(2)
    transpose(2)
    start_w(198, 2)
    wait_g(3)
    transpose(3)
    start_w(199, 3)
    wait_w(196, 0)
    wait_w(197, 1)
    wait_w(198, 2)
    wait_w(199, 3)

  return gather_kernel


_kernel_call = _make_kernel()


@jax.jit
def kernel(x, W):
  out_t = _kernel_call(x.astype(jnp.int32), W)
  return jnp.transpose(out_t, (2, 0, 1))
